# baseline (speedup 1.0000x reference)
import functools

import jax
import jax.numpy as jnp
from jax import lax
from jax.experimental import pallas as pl
from jax.experimental.pallas import tpu as pltpu
from jax.experimental.pallas import tpu_sc as plsc

N = 10000
NPAD = 10240
E = 320000
C = 128
NCORES = 2
NSUB = 16
NTILES = NCORES * NSUB
NBUF = 4
CHUNKS = 80
EPAD = NTILES * CHUNKS * C
ROWS_PER_SUB = NPAD // NSUB
IN_CH, HID, OUT_CH = 128, 64, 128


def _make_sc_agg(d):
  mesh = plsc.VectorSubcoreMesh(core_axis_name="c", subcore_axis_name="s")

  @functools.partial(
      pl.kernel,
      out_type=jax.ShapeDtypeStruct((NCORES, NPAD, d), jnp.float32),
      mesh=mesh,
      compiler_params=pltpu.CompilerParams(use_tc_tiling_on_sc=False),
      scratch_types=[
          pltpu.VMEM((CHUNKS, C), jnp.int32),
          pltpu.VMEM((CHUNKS, C), jnp.int32),
          *([pltpu.VMEM((C, d), jnp.float32)] * NBUF),
          pltpu.VMEM_SHARED((NPAD, d), jnp.float32),
          *([pltpu.SemaphoreType.DMA] * NBUF),
      ],
  )
  def agg(table_hbm, src_hbm, dst_hbm, zeros_hbm, out_hbm,
          src_v, dst_v, *rest):
    bufs = rest[:NBUF]
    acc_sh = rest[NBUF]
    sg = rest[NBUF + 1:]
    c = lax.axis_index("c")
    s = lax.axis_index("s")
    w = c * NSUB + s

    pltpu.sync_copy(zeros_hbm, bufs[0])
    row0 = s * ROWS_PER_SUB
    for k in range(ROWS_PER_SUB // C):
      pltpu.sync_copy(bufs[0], acc_sh.at[pl.ds(row0 + k * C, C)])

    pltpu.sync_copy(src_hbm.at[w], src_v)
    pltpu.sync_copy(dst_hbm.at[w], dst_v)
    plsc.subcore_barrier()

    for b in range(NBUF):
      pltpu.async_copy(table_hbm.at[src_v.at[b]], bufs[b], sg[b])

    @pl.loop(0, CHUNKS, step=NBUF)
    def _(j):
      for b in range(NBUF):
        jj = j + b
        pltpu.make_async_copy(table_hbm.at[src_v.at[0]], bufs[b], sg[b]).wait()
        pltpu.sync_copy(bufs[b], acc_sh.at[dst_v.at[jj]], add=True)

        @pl.when(jj + NBUF < CHUNKS)
        def _():
          pltpu.async_copy(table_hbm.at[src_v.at[jj + NBUF]], bufs[b], sg[b])

    plsc.subcore_barrier()

    for k in range(ROWS_PER_SUB // C):
      sl = pl.ds(row0 + k * C, C)
      pltpu.sync_copy(acc_sh.at[sl], bufs[0])
      pltpu.sync_copy(bufs[0], out_hbm.at[c, sl])

  return agg


def _make_sc_deg():
  mesh = plsc.VectorSubcoreMesh(core_axis_name="c", subcore_axis_name="s")

  @functools.partial(
      pl.kernel,
      out_type=jax.ShapeDtypeStruct((NCORES, NPAD, 16), jnp.float32),
      mesh=mesh,
      compiler_params=pltpu.CompilerParams(use_tc_tiling_on_sc=False),
      scratch_types=[
          pltpu.VMEM((CHUNKS, C), jnp.int32),
          pltpu.VMEM((C, 16), jnp.float32),
          pltpu.VMEM((C, 16), jnp.float32),
          pltpu.VMEM_SHARED((NPAD, 16), jnp.float32),
          *([pltpu.SemaphoreType.DMA] * NBUF),
      ],
  )
  def deg(ones_hbm, zeros_hbm, dst_hbm, out_hbm, dst_v, ones_v, buf_v,
          acc_sh, *ss):
    c = lax.axis_index("c")
    s = lax.axis_index("s")
    w = c * NSUB + s

    pltpu.sync_copy(zeros_hbm, buf_v)
    row0 = s * ROWS_PER_SUB
    for k in range(ROWS_PER_SUB // C):
      pltpu.sync_copy(buf_v, acc_sh.at[pl.ds(row0 + k * C, C)])
    pltpu.sync_copy(ones_hbm, ones_v)
    pltpu.sync_copy(dst_hbm.at[w], dst_v)
    plsc.subcore_barrier()

    @pl.loop(0, CHUNKS)
    def _(j):
      pltpu.sync_copy(ones_v, acc_sh.at[dst_v.at[j]], add=True)

    plsc.subcore_barrier()

    for k in range(ROWS_PER_SUB // C):
      sl = pl.ds(row0 + k * C, C)
      pltpu.sync_copy(acc_sh.at[sl], buf_v)
      pltpu.sync_copy(buf_v, out_hbm.at[c, sl])

  return deg


_sc_agg = _make_sc_agg(HID)
_sc_deg = _make_sc_deg()


_BM = 1024
_GRID = NPAD // _BM


def _mm_body(x_ref, w_ref, o_ref):
  o_ref[...] = jnp.dot(x_ref[...], w_ref[...],
                       preferred_element_type=jnp.float32)


def _tc_matmul(x, w):
  m, k = x.shape
  n = w.shape[1]
  return pl.pallas_call(
      _mm_body,
      grid=(m // _BM,),
      in_specs=[pl.BlockSpec((_BM, k), lambda i: (i, 0)),
                pl.BlockSpec((k, n), lambda i: (0, 0))],
      out_specs=pl.BlockSpec((_BM, n), lambda i: (i, 0)),
      out_shape=jax.ShapeDtypeStruct((m, n), jnp.float32),
  )(x, w)


def _dinv_scale_body(degp_ref, h_ref, dinv_ref, g_ref, i_ref=None):
  del i_ref
  i = pl.program_id(0)
  deg = degp_ref[0, :, 0:1] + degp_ref[1, :, 0:1] + 1.0
  rid = lax.broadcasted_iota(jnp.int32, (_BM, 1), 0) + i * _BM
  dinv = jnp.where(rid < N, lax.rsqrt(deg), 0.0)
  dinv_ref[...] = dinv
  g_ref[...] = h_ref[...] * dinv


def _tc_dinv_scale(degp, h):
  return pl.pallas_call(
      _dinv_scale_body,
      grid=(_GRID,),
      in_specs=[pl.BlockSpec((NCORES, _BM, 16), lambda i: (0, i, 0)),
                pl.BlockSpec((_BM, HID), lambda i: (i, 0))],
      out_specs=[pl.BlockSpec((_BM, 1), lambda i: (i, 0)),
                 pl.BlockSpec((_BM, HID), lambda i: (i, 0))],
      out_shape=[jax.ShapeDtypeStruct((NPAD, 1), jnp.float32),
                 jax.ShapeDtypeStruct((NPAD, HID), jnp.float32)],
  )(degp, h)


def _mid_body(p_ref, g_ref, dinv_ref, b_ref, w_ref, g2a_ref, g2b_ref):
  acc = p_ref[0] + p_ref[1] + g_ref[...]
  z = jax.nn.relu(dinv_ref[...] * acc + b_ref[...])
  g2 = dinv_ref[...] * jnp.dot(z, w_ref[...],
                               preferred_element_type=jnp.float32)
  g2a_ref[...] = g2[:, :HID]
  g2b_ref[...] = g2[:, HID:]


def _tc_mid(p, g, dinv, b, w):
  return pl.pallas_call(
      _mid_body,
      grid=(_GRID,),
      in_specs=[pl.BlockSpec((NCORES, _BM, HID), lambda i: (0, i, 0)),
                pl.BlockSpec((_BM, HID), lambda i: (i, 0)),
                pl.BlockSpec((_BM, 1), lambda i: (i, 0)),
                pl.BlockSpec((1, HID), lambda i: (0, 0)),
                pl.BlockSpec((HID, OUT_CH), lambda i: (0, 0))],
      out_specs=[pl.BlockSpec((_BM, HID), lambda i: (i, 0)),
                 pl.BlockSpec((_BM, HID), lambda i: (i, 0))],
      out_shape=[jax.ShapeDtypeStruct((NPAD, HID), jnp.float32),
                 jax.ShapeDtypeStruct((NPAD, HID), jnp.float32)],
  )(p, g, dinv, b, w)


def _final_body(pa_ref, pb_ref, ga_ref, gb_ref, dinv_ref, b_ref, o_ref):
  ha = pa_ref[0] + pa_ref[1] + ga_ref[...]
  hb = pb_ref[0] + pb_ref[1] + gb_ref[...]
  acc = jnp.concatenate([ha, hb], axis=1)
  o_ref[...] = jax.nn.relu(dinv_ref[...] * acc + b_ref[...])


def _tc_final(pa, pb, ga, gb, dinv, b):
  return pl.pallas_call(
      _final_body,
      grid=(_GRID,),
      in_specs=[pl.BlockSpec((NCORES, _BM, HID), lambda i: (0, i, 0)),
                pl.BlockSpec((NCORES, _BM, HID), lambda i: (0, i, 0)),
                pl.BlockSpec((_BM, HID), lambda i: (i, 0)),
                pl.BlockSpec((_BM, HID), lambda i: (i, 0)),
                pl.BlockSpec((_BM, 1), lambda i: (i, 0)),
                pl.BlockSpec((1, OUT_CH), lambda i: (0, 0))],
      out_specs=pl.BlockSpec((_BM, OUT_CH), lambda i: (i, 0)),
      out_shape=jax.ShapeDtypeStruct((NPAD, OUT_CH), jnp.float32),
  )(pa, pb, ga, gb, dinv, b)


def kernel(x, edge_index, W1, b1, W2, b2):
  src = edge_index[0].astype(jnp.int32)
  dst = edge_index[1].astype(jnp.int32)
  pad = jnp.full((EPAD - E,), N, jnp.int32)
  src_p = jnp.concatenate([src, pad]).reshape(NTILES, CHUNKS, C)
  dst_p = jnp.concatenate([dst, pad]).reshape(NTILES, CHUNKS, C)
  x_pad = jnp.zeros((NPAD, IN_CH), jnp.float32).at[:N].set(x)
  z16 = jnp.zeros((C, 16), jnp.float32)
  z64 = jnp.zeros((C, HID), jnp.float32)
  ones16 = jnp.ones((C, 16), jnp.float32)

  degp = _sc_deg(ones16, z16, dst_p)
  h1 = _tc_matmul(x_pad, W1)

  dinv, g1 = _tc_dinv_scale(degp, h1)
  p1 = _sc_agg(g1, src_p, dst_p, z64)
  g2a, g2b = _tc_mid(p1, g1, dinv, b1.reshape(1, HID), W2)
  p2a = _sc_agg(g2a, src_p, dst_p, z64)
  p2b = _sc_agg(g2b, src_p, dst_p, z64)
  out = _tc_final(p2a, p2b, g2a, g2b, dinv, b2.reshape(1, OUT_CH))
  return out[:N]


# speedup vs baseline: 1.0010x; 1.0010x over previous
import functools

import jax
import jax.numpy as jnp
from jax import lax
from jax.experimental import pallas as pl
from jax.experimental.pallas import tpu as pltpu
from jax.experimental.pallas import tpu_sc as plsc

N = 10000
NPAD = 10240
E = 320000
C = 128
NCORES = 2
NSUB = 16
NTILES = NCORES * NSUB
NBUF = 4
CHUNKS = 80
EPAD = NTILES * CHUNKS * C
ROWS_PER_SUB = NPAD // NSUB
IN_CH, HID, OUT_CH = 128, 64, 128


def _make_sc_agg(d):
  mesh = plsc.VectorSubcoreMesh(core_axis_name="c", subcore_axis_name="s")

  @functools.partial(
      pl.kernel,
      out_type=jax.ShapeDtypeStruct((NCORES, NPAD, d), jnp.float32),
      mesh=mesh,
      compiler_params=pltpu.CompilerParams(use_tc_tiling_on_sc=False),
      scratch_types=[
          pltpu.VMEM((CHUNKS, C), jnp.int32),
          pltpu.VMEM((CHUNKS, C), jnp.int32),
          *([pltpu.VMEM((C, d), jnp.float32)] * NBUF),
          pltpu.VMEM_SHARED((NPAD, d), jnp.float32),
          *([pltpu.SemaphoreType.DMA] * (2 * NBUF)),
      ],
  )
  def agg(table_hbm, src_hbm, dst_hbm, zeros_hbm, out_hbm,
          src_v, dst_v, *rest):
    bufs = rest[:NBUF]
    acc_sh = rest[NBUF]
    sg = rest[NBUF + 1:NBUF + 1 + NBUF]
    ss = rest[NBUF + 1 + NBUF:]
    c = lax.axis_index("c")
    s = lax.axis_index("s")
    w = c * NSUB + s

    pltpu.sync_copy(zeros_hbm, bufs[0])
    row0 = s * ROWS_PER_SUB
    for k in range(ROWS_PER_SUB // C):
      pltpu.sync_copy(bufs[0], acc_sh.at[pl.ds(row0 + k * C, C)])

    pltpu.sync_copy(src_hbm.at[w], src_v)
    pltpu.sync_copy(dst_hbm.at[w], dst_v)
    plsc.subcore_barrier()

    for b in range(NBUF):
      pltpu.async_copy(table_hbm.at[src_v.at[b]], bufs[b], sg[b])

    @pl.loop(0, CHUNKS, step=NBUF)
    def _(j):
      for b in range(NBUF):
        jj = j + b
        pltpu.make_async_copy(table_hbm.at[src_v.at[0]], bufs[b], sg[b]).wait()
        pltpu.async_copy(bufs[b], acc_sh.at[dst_v.at[jj]], ss[b], add=True)
        b3 = (b + 2) % NBUF
        rc = jj + 2
        cond = (j > 0) if b < 2 else (rc < CHUNKS)

        @pl.when(cond)
        def _():
          pltpu.make_async_copy(bufs[b3], acc_sh.at[dst_v.at[0]], ss[b3]).wait()
          pltpu.async_copy(table_hbm.at[src_v.at[rc]], bufs[b3], sg[b3])

    for b in range(NBUF):
      pltpu.make_async_copy(bufs[b], acc_sh.at[dst_v.at[0]], ss[b]).wait()
    plsc.subcore_barrier()

    for k in range(ROWS_PER_SUB // C):
      sl = pl.ds(row0 + k * C, C)
      pltpu.sync_copy(acc_sh.at[sl], bufs[0])
      pltpu.sync_copy(bufs[0], out_hbm.at[c, sl])

  return agg


def _make_sc_deg():
  mesh = plsc.VectorSubcoreMesh(core_axis_name="c", subcore_axis_name="s")

  @functools.partial(
      pl.kernel,
      out_type=jax.ShapeDtypeStruct((NCORES, NPAD, 16), jnp.float32),
      mesh=mesh,
      compiler_params=pltpu.CompilerParams(use_tc_tiling_on_sc=False),
      scratch_types=[
          pltpu.VMEM((CHUNKS, C), jnp.int32),
          pltpu.VMEM((C, 16), jnp.float32),
          pltpu.VMEM((C, 16), jnp.float32),
          pltpu.VMEM_SHARED((NPAD, 16), jnp.float32),
          *([pltpu.SemaphoreType.DMA] * NBUF),
      ],
  )
  def deg(ones_hbm, zeros_hbm, dst_hbm, out_hbm, dst_v, ones_v, buf_v,
          acc_sh, *ss):
    c = lax.axis_index("c")
    s = lax.axis_index("s")
    w = c * NSUB + s

    pltpu.sync_copy(zeros_hbm, buf_v)
    row0 = s * ROWS_PER_SUB
    for k in range(ROWS_PER_SUB // C):
      pltpu.sync_copy(buf_v, acc_sh.at[pl.ds(row0 + k * C, C)])
    pltpu.sync_copy(ones_hbm, ones_v)
    pltpu.sync_copy(dst_hbm.at[w], dst_v)
    plsc.subcore_barrier()

    @pl.loop(0, CHUNKS)
    def _(j):
      pltpu.sync_copy(ones_v, acc_sh.at[dst_v.at[j]], add=True)

    plsc.subcore_barrier()

    for k in range(ROWS_PER_SUB // C):
      sl = pl.ds(row0 + k * C, C)
      pltpu.sync_copy(acc_sh.at[sl], buf_v)
      pltpu.sync_copy(buf_v, out_hbm.at[c, sl])

  return deg


_sc_agg = _make_sc_agg(HID)
_sc_deg = _make_sc_deg()


_BM = 1024
_GRID = NPAD // _BM


def _mm_body(x_ref, w_ref, o_ref):
  o_ref[...] = jnp.dot(x_ref[...], w_ref[...],
                       preferred_element_type=jnp.float32)


def _tc_matmul(x, w):
  m, k = x.shape
  n = w.shape[1]
  return pl.pallas_call(
      _mm_body,
      grid=(m // _BM,),
      in_specs=[pl.BlockSpec((_BM, k), lambda i: (i, 0)),
                pl.BlockSpec((k, n), lambda i: (0, 0))],
      out_specs=pl.BlockSpec((_BM, n), lambda i: (i, 0)),
      out_shape=jax.ShapeDtypeStruct((m, n), jnp.float32),
  )(x, w)


def _dinv_scale_body(degp_ref, h_ref, dinv_ref, g_ref, i_ref=None):
  del i_ref
  i = pl.program_id(0)
  deg = degp_ref[0, :, 0:1] + degp_ref[1, :, 0:1] + 1.0
  rid = lax.broadcasted_iota(jnp.int32, (_BM, 1), 0) + i * _BM
  dinv = jnp.where(rid < N, lax.rsqrt(deg), 0.0)
  dinv_ref[...] = dinv
  g_ref[...] = h_ref[...] * dinv


def _tc_dinv_scale(degp, h):
  return pl.pallas_call(
      _dinv_scale_body,
      grid=(_GRID,),
      in_specs=[pl.BlockSpec((NCORES, _BM, 16), lambda i: (0, i, 0)),
                pl.BlockSpec((_BM, HID), lambda i: (i, 0))],
      out_specs=[pl.BlockSpec((_BM, 1), lambda i: (i, 0)),
                 pl.BlockSpec((_BM, HID), lambda i: (i, 0))],
      out_shape=[jax.ShapeDtypeStruct((NPAD, 1), jnp.float32),
                 jax.ShapeDtypeStruct((NPAD, HID), jnp.float32)],
  )(degp, h)


def _mid_body(p_ref, g_ref, dinv_ref, b_ref, w_ref, g2a_ref, g2b_ref):
  acc = p_ref[0] + p_ref[1] + g_ref[...]
  z = jax.nn.relu(dinv_ref[...] * acc + b_ref[...])
  g2 = dinv_ref[...] * jnp.dot(z, w_ref[...],
                               preferred_element_type=jnp.float32)
  g2a_ref[...] = g2[:, :HID]
  g2b_ref[...] = g2[:, HID:]


def _tc_mid(p, g, dinv, b, w):
  return pl.pallas_call(
      _mid_body,
      grid=(_GRID,),
      in_specs=[pl.BlockSpec((NCORES, _BM, HID), lambda i: (0, i, 0)),
                pl.BlockSpec((_BM, HID), lambda i: (i, 0)),
                pl.BlockSpec((_BM, 1), lambda i: (i, 0)),
                pl.BlockSpec((1, HID), lambda i: (0, 0)),
                pl.BlockSpec((HID, OUT_CH), lambda i: (0, 0))],
      out_specs=[pl.BlockSpec((_BM, HID), lambda i: (i, 0)),
                 pl.BlockSpec((_BM, HID), lambda i: (i, 0))],
      out_shape=[jax.ShapeDtypeStruct((NPAD, HID), jnp.float32),
                 jax.ShapeDtypeStruct((NPAD, HID), jnp.float32)],
  )(p, g, dinv, b, w)


def _final_body(pa_ref, pb_ref, ga_ref, gb_ref, dinv_ref, b_ref, o_ref):
  ha = pa_ref[0] + pa_ref[1] + ga_ref[...]
  hb = pb_ref[0] + pb_ref[1] + gb_ref[...]
  acc = jnp.concatenate([ha, hb], axis=1)
  o_ref[...] = jax.nn.relu(dinv_ref[...] * acc + b_ref[...])


def _tc_final(pa, pb, ga, gb, dinv, b):
  return pl.pallas_call(
      _final_body,
      grid=(_GRID,),
      in_specs=[pl.BlockSpec((NCORES, _BM, HID), lambda i: (0, i, 0)),
                pl.BlockSpec((NCORES, _BM, HID), lambda i: (0, i, 0)),
                pl.BlockSpec((_BM, HID), lambda i: (i, 0)),
                pl.BlockSpec((_BM, HID), lambda i: (i, 0)),
                pl.BlockSpec((_BM, 1), lambda i: (i, 0)),
                pl.BlockSpec((1, OUT_CH), lambda i: (0, 0))],
      out_specs=pl.BlockSpec((_BM, OUT_CH), lambda i: (i, 0)),
      out_shape=jax.ShapeDtypeStruct((NPAD, OUT_CH), jnp.float32),
  )(pa, pb, ga, gb, dinv, b)


def kernel(x, edge_index, W1, b1, W2, b2):
  src = edge_index[0].astype(jnp.int32)
  dst = edge_index[1].astype(jnp.int32)
  pad = jnp.full((EPAD - E,), N, jnp.int32)
  src_p = jnp.concatenate([src, pad]).reshape(NTILES, CHUNKS, C)
  dst_p = jnp.concatenate([dst, pad]).reshape(NTILES, CHUNKS, C)
  x_pad = jnp.zeros((NPAD, IN_CH), jnp.float32).at[:N].set(x)
  z16 = jnp.zeros((C, 16), jnp.float32)
  z64 = jnp.zeros((C, HID), jnp.float32)
  ones16 = jnp.ones((C, 16), jnp.float32)

  degp = _sc_deg(ones16, z16, dst_p)
  h1 = _tc_matmul(x_pad, W1)

  dinv, g1 = _tc_dinv_scale(degp, h1)
  p1 = _sc_agg(g1, src_p, dst_p, z64)
  g2a, g2b = _tc_mid(p1, g1, dinv, b1.reshape(1, HID), W2)
  p2a = _sc_agg(g2a, src_p, dst_p, z64)
  p2b = _sc_agg(g2b, src_p, dst_p, z64)
  out = _tc_final(p2a, p2b, g2a, g2b, dinv, b2.reshape(1, OUT_CH))
  return out[:N]


# speedup vs baseline: 1.8429x; 1.8411x over previous
import functools

import jax
import jax.numpy as jnp
from jax import lax
from jax.experimental import pallas as pl
from jax.experimental.pallas import tpu as pltpu
from jax.experimental.pallas import tpu_sc as plsc

N = 10000
NPAD = 10240
E = 320000
C = 128
NCORES = 2
NSUB = 16
NTILES = NCORES * NSUB
NBUF = 4
CHUNKS = 80
EPAD = NTILES * CHUNKS * C
ROWS_PER_SUB = NPAD // NSUB
IN_CH, HID, OUT_CH = 128, 64, 128


def _make_sc_agg(d):
  mesh = plsc.VectorSubcoreMesh(core_axis_name="c", subcore_axis_name="s")

  @functools.partial(
      pl.kernel,
      out_type=jax.ShapeDtypeStruct((NCORES, NPAD, d), jnp.float32),
      mesh=mesh,
      compiler_params=pltpu.CompilerParams(use_tc_tiling_on_sc=False),
      scratch_types=[
          pltpu.VMEM((CHUNKS, C), jnp.int32),
          pltpu.VMEM((CHUNKS, C), jnp.int32),
          pltpu.VMEM((C, d), jnp.float32),
          pltpu.VMEM_SHARED((NPAD, d), jnp.float32),
          pltpu.VMEM_SHARED((NPAD, d), jnp.float32),
      ],
  )
  def agg(table_hbm, src_hbm, dst_hbm, zeros_hbm, out_hbm,
          src_v, dst_v, buf_v, table_sh, acc_sh):
    c = lax.axis_index("c")
    s = lax.axis_index("s")
    w = c * NSUB + s
    row0 = s * ROWS_PER_SUB

    pltpu.sync_copy(zeros_hbm, buf_v)
    for k in range(ROWS_PER_SUB // C):
      pltpu.sync_copy(buf_v, acc_sh.at[pl.ds(row0 + k * C, C)])
    for k in range(ROWS_PER_SUB // C):
      sl = pl.ds(row0 + k * C, C)
      pltpu.sync_copy(table_hbm.at[sl], buf_v)
      pltpu.sync_copy(buf_v, table_sh.at[sl])

    pltpu.sync_copy(src_hbm.at[w], src_v)
    pltpu.sync_copy(dst_hbm.at[w], dst_v)
    plsc.subcore_barrier()

    @pl.loop(0, CHUNKS)
    def _(j):
      pltpu.sync_copy(table_sh.at[src_v.at[j]], buf_v)
      pltpu.sync_copy(buf_v, acc_sh.at[dst_v.at[j]], add=True)

    plsc.subcore_barrier()

    for k in range(ROWS_PER_SUB // C):
      sl = pl.ds(row0 + k * C, C)
      pltpu.sync_copy(acc_sh.at[sl], buf_v)
      pltpu.sync_copy(buf_v, out_hbm.at[c, sl])

  return agg


def _make_sc_deg():
  mesh = plsc.VectorSubcoreMesh(core_axis_name="c", subcore_axis_name="s")

  @functools.partial(
      pl.kernel,
      out_type=jax.ShapeDtypeStruct((NCORES, NPAD, 16), jnp.float32),
      mesh=mesh,
      compiler_params=pltpu.CompilerParams(use_tc_tiling_on_sc=False),
      scratch_types=[
          pltpu.VMEM((CHUNKS, C), jnp.int32),
          pltpu.VMEM((C, 16), jnp.float32),
          pltpu.VMEM((C, 16), jnp.float32),
          pltpu.VMEM_SHARED((NPAD, 16), jnp.float32),
          *([pltpu.SemaphoreType.DMA] * NBUF),
      ],
  )
  def deg(ones_hbm, zeros_hbm, dst_hbm, out_hbm, dst_v, ones_v, buf_v,
          acc_sh, *ss):
    c = lax.axis_index("c")
    s = lax.axis_index("s")
    w = c * NSUB + s

    pltpu.sync_copy(zeros_hbm, buf_v)
    row0 = s * ROWS_PER_SUB
    for k in range(ROWS_PER_SUB // C):
      pltpu.sync_copy(buf_v, acc_sh.at[pl.ds(row0 + k * C, C)])
    pltpu.sync_copy(ones_hbm, ones_v)
    pltpu.sync_copy(dst_hbm.at[w], dst_v)
    plsc.subcore_barrier()

    @pl.loop(0, CHUNKS)
    def _(j):
      pltpu.sync_copy(ones_v, acc_sh.at[dst_v.at[j]], add=True)

    plsc.subcore_barrier()

    for k in range(ROWS_PER_SUB // C):
      sl = pl.ds(row0 + k * C, C)
      pltpu.sync_copy(acc_sh.at[sl], buf_v)
      pltpu.sync_copy(buf_v, out_hbm.at[c, sl])

  return deg


_sc_agg = _make_sc_agg(HID)
_sc_deg = _make_sc_deg()


_BM = 1024
_GRID = NPAD // _BM


def _mm_body(x_ref, w_ref, o_ref):
  o_ref[...] = jnp.dot(x_ref[...], w_ref[...],
                       preferred_element_type=jnp.float32)


def _tc_matmul(x, w):
  m, k = x.shape
  n = w.shape[1]
  return pl.pallas_call(
      _mm_body,
      grid=(m // _BM,),
      in_specs=[pl.BlockSpec((_BM, k), lambda i: (i, 0)),
                pl.BlockSpec((k, n), lambda i: (0, 0))],
      out_specs=pl.BlockSpec((_BM, n), lambda i: (i, 0)),
      out_shape=jax.ShapeDtypeStruct((m, n), jnp.float32),
  )(x, w)


def _dinv_scale_body(degp_ref, h_ref, dinv_ref, g_ref, i_ref=None):
  del i_ref
  i = pl.program_id(0)
  deg = degp_ref[0, :, 0:1] + degp_ref[1, :, 0:1] + 1.0
  rid = lax.broadcasted_iota(jnp.int32, (_BM, 1), 0) + i * _BM
  dinv = jnp.where(rid < N, lax.rsqrt(deg), 0.0)
  dinv_ref[...] = dinv
  g_ref[...] = h_ref[...] * dinv


def _tc_dinv_scale(degp, h):
  return pl.pallas_call(
      _dinv_scale_body,
      grid=(_GRID,),
      in_specs=[pl.BlockSpec((NCORES, _BM, 16), lambda i: (0, i, 0)),
                pl.BlockSpec((_BM, HID), lambda i: (i, 0))],
      out_specs=[pl.BlockSpec((_BM, 1), lambda i: (i, 0)),
                 pl.BlockSpec((_BM, HID), lambda i: (i, 0))],
      out_shape=[jax.ShapeDtypeStruct((NPAD, 1), jnp.float32),
                 jax.ShapeDtypeStruct((NPAD, HID), jnp.float32)],
  )(degp, h)


def _mid_body(p_ref, g_ref, dinv_ref, b_ref, w_ref, g2a_ref, g2b_ref):
  acc = p_ref[0] + p_ref[1] + g_ref[...]
  z = jax.nn.relu(dinv_ref[...] * acc + b_ref[...])
  g2 = dinv_ref[...] * jnp.dot(z, w_ref[...],
                               preferred_element_type=jnp.float32)
  g2a_ref[...] = g2[:, :HID]
  g2b_ref[...] = g2[:, HID:]


def _tc_mid(p, g, dinv, b, w):
  return pl.pallas_call(
      _mid_body,
      grid=(_GRID,),
      in_specs=[pl.BlockSpec((NCORES, _BM, HID), lambda i: (0, i, 0)),
                pl.BlockSpec((_BM, HID), lambda i: (i, 0)),
                pl.BlockSpec((_BM, 1), lambda i: (i, 0)),
                pl.BlockSpec((1, HID), lambda i: (0, 0)),
                pl.BlockSpec((HID, OUT_CH), lambda i: (0, 0))],
      out_specs=[pl.BlockSpec((_BM, HID), lambda i: (i, 0)),
                 pl.BlockSpec((_BM, HID), lambda i: (i, 0))],
      out_shape=[jax.ShapeDtypeStruct((NPAD, HID), jnp.float32),
                 jax.ShapeDtypeStruct((NPAD, HID), jnp.float32)],
  )(p, g, dinv, b, w)


def _final_body(pa_ref, pb_ref, ga_ref, gb_ref, dinv_ref, b_ref, o_ref):
  ha = pa_ref[0] + pa_ref[1] + ga_ref[...]
  hb = pb_ref[0] + pb_ref[1] + gb_ref[...]
  acc = jnp.concatenate([ha, hb], axis=1)
  o_ref[...] = jax.nn.relu(dinv_ref[...] * acc + b_ref[...])


def _tc_final(pa, pb, ga, gb, dinv, b):
  return pl.pallas_call(
      _final_body,
      grid=(_GRID,),
      in_specs=[pl.BlockSpec((NCORES, _BM, HID), lambda i: (0, i, 0)),
                pl.BlockSpec((NCORES, _BM, HID), lambda i: (0, i, 0)),
                pl.BlockSpec((_BM, HID), lambda i: (i, 0)),
                pl.BlockSpec((_BM, HID), lambda i: (i, 0)),
                pl.BlockSpec((_BM, 1), lambda i: (i, 0)),
                pl.BlockSpec((1, OUT_CH), lambda i: (0, 0))],
      out_specs=pl.BlockSpec((_BM, OUT_CH), lambda i: (i, 0)),
      out_shape=jax.ShapeDtypeStruct((NPAD, OUT_CH), jnp.float32),
  )(pa, pb, ga, gb, dinv, b)


def kernel(x, edge_index, W1, b1, W2, b2):
  src = edge_index[0].astype(jnp.int32)
  dst = edge_index[1].astype(jnp.int32)
  pad = jnp.full((EPAD - E,), N, jnp.int32)
  src_p = jnp.concatenate([src, pad]).reshape(NTILES, CHUNKS, C)
  dst_p = jnp.concatenate([dst, pad]).reshape(NTILES, CHUNKS, C)
  x_pad = jnp.zeros((NPAD, IN_CH), jnp.float32).at[:N].set(x)
  z16 = jnp.zeros((C, 16), jnp.float32)
  z64 = jnp.zeros((C, HID), jnp.float32)
  ones16 = jnp.ones((C, 16), jnp.float32)

  degp = _sc_deg(ones16, z16, dst_p)
  h1 = _tc_matmul(x_pad, W1)

  dinv, g1 = _tc_dinv_scale(degp, h1)
  p1 = _sc_agg(g1, src_p, dst_p, z64)
  g2a, g2b = _tc_mid(p1, g1, dinv, b1.reshape(1, HID), W2)
  p2a = _sc_agg(g2a, src_p, dst_p, z64)
  p2b = _sc_agg(g2b, src_p, dst_p, z64)
  out = _tc_final(p2a, p2b, g2a, g2b, dinv, b2.reshape(1, OUT_CH))
  return out[:N]


# speedup vs baseline: 1.9576x; 1.0623x over previous
import functools

import jax
import jax.numpy as jnp
from jax import lax
from jax.experimental import pallas as pl
from jax.experimental.pallas import tpu as pltpu
from jax.experimental.pallas import tpu_sc as plsc

N = 10000
NPAD = 10240
E = 320000
C = 128
NCORES = 2
NSUB = 16
NTILES = NCORES * NSUB
NBUF = 4
CHUNKS = 80
EPAD = NTILES * CHUNKS * C
ROWS_PER_SUB = NPAD // NSUB
IN_CH, HID, OUT_CH = 128, 64, 128


def _make_sc_agg(d):
  mesh = plsc.VectorSubcoreMesh(core_axis_name="c", subcore_axis_name="s")

  @functools.partial(
      pl.kernel,
      out_type=jax.ShapeDtypeStruct((NCORES, NPAD, d), jnp.float32),
      mesh=mesh,
      compiler_params=pltpu.CompilerParams(use_tc_tiling_on_sc=False),
      scratch_types=[
          pltpu.VMEM((CHUNKS, C), jnp.int32),
          pltpu.VMEM((CHUNKS, C), jnp.int32),
          pltpu.VMEM((C, d), jnp.float32),
          pltpu.VMEM_SHARED((NPAD, d), jnp.float32),
          pltpu.VMEM_SHARED((NPAD, d), jnp.float32),
      ],
  )
  def agg(table_hbm, src_hbm, dst_hbm, zeros_hbm, out_hbm,
          src_v, dst_v, buf_v, table_sh, acc_sh):
    c = lax.axis_index("c")
    s = lax.axis_index("s")
    w = c * NSUB + s
    row0 = s * ROWS_PER_SUB

    pltpu.sync_copy(zeros_hbm, buf_v)
    for k in range(ROWS_PER_SUB // C):
      pltpu.sync_copy(buf_v, acc_sh.at[pl.ds(row0 + k * C, C)])
    for k in range(ROWS_PER_SUB // C):
      sl = pl.ds(row0 + k * C, C)
      pltpu.sync_copy(table_hbm.at[sl], buf_v)
      pltpu.sync_copy(buf_v, table_sh.at[sl])

    pltpu.sync_copy(src_hbm.at[w], src_v)
    pltpu.sync_copy(dst_hbm.at[w], dst_v)
    plsc.subcore_barrier()

    @pl.loop(0, CHUNKS)
    def _(j):
      pltpu.sync_copy(table_sh.at[src_v.at[j]], buf_v)
      pltpu.sync_copy(buf_v, acc_sh.at[dst_v.at[j]], add=True)

    plsc.subcore_barrier()

    for k in range(ROWS_PER_SUB // C):
      sl = pl.ds(row0 + k * C, C)
      pltpu.sync_copy(acc_sh.at[sl], buf_v)
      pltpu.sync_copy(buf_v, out_hbm.at[c, sl])

  return agg


CHUNKS2 = CHUNKS * NCORES


def _make_sc_agg_colsplit(d):
  mesh = plsc.VectorSubcoreMesh(core_axis_name="c", subcore_axis_name="s")

  @functools.partial(
      pl.kernel,
      out_type=jax.ShapeDtypeStruct((NCORES, NPAD, d), jnp.float32),
      mesh=mesh,
      compiler_params=pltpu.CompilerParams(use_tc_tiling_on_sc=False),
      scratch_types=[
          pltpu.VMEM((CHUNKS2, C), jnp.int32),
          pltpu.VMEM((CHUNKS2, C), jnp.int32),
          pltpu.VMEM((C, d), jnp.float32),
          pltpu.VMEM_SHARED((NPAD, d), jnp.float32),
          pltpu.VMEM_SHARED((NPAD, d), jnp.float32),
      ],
  )
  def agg2(ta_hbm, tb_hbm, src_hbm, dst_hbm, zeros_hbm, out_hbm,
           src_v, dst_v, buf_v, table_sh, acc_sh):
    c = lax.axis_index("c")
    s = lax.axis_index("s")
    row0 = s * ROWS_PER_SUB

    pltpu.sync_copy(zeros_hbm, buf_v)
    for k in range(ROWS_PER_SUB // C):
      pltpu.sync_copy(buf_v, acc_sh.at[pl.ds(row0 + k * C, C)])
    for k in range(ROWS_PER_SUB // C):
      sl = pl.ds(row0 + k * C, C)

      @pl.when(c == 0)
      def _():
        pltpu.sync_copy(ta_hbm.at[sl], buf_v)

      @pl.when(c == 1)
      def _():
        pltpu.sync_copy(tb_hbm.at[sl], buf_v)

      pltpu.sync_copy(buf_v, table_sh.at[sl])

    pltpu.sync_copy(src_hbm.at[s], src_v)
    pltpu.sync_copy(dst_hbm.at[s], dst_v)
    plsc.subcore_barrier()

    @pl.loop(0, CHUNKS2)
    def _(j):
      pltpu.sync_copy(table_sh.at[src_v.at[j]], buf_v)
      pltpu.sync_copy(buf_v, acc_sh.at[dst_v.at[j]], add=True)

    plsc.subcore_barrier()

    for k in range(ROWS_PER_SUB // C):
      sl = pl.ds(row0 + k * C, C)
      pltpu.sync_copy(acc_sh.at[sl], buf_v)
      pltpu.sync_copy(buf_v, out_hbm.at[c, sl])

  return agg2


def _make_sc_deg():
  mesh = plsc.VectorSubcoreMesh(core_axis_name="c", subcore_axis_name="s")

  @functools.partial(
      pl.kernel,
      out_type=jax.ShapeDtypeStruct((NCORES, NPAD, 16), jnp.float32),
      mesh=mesh,
      compiler_params=pltpu.CompilerParams(use_tc_tiling_on_sc=False),
      scratch_types=[
          pltpu.VMEM((CHUNKS, C), jnp.int32),
          pltpu.VMEM((C, 16), jnp.float32),
          pltpu.VMEM((C, 16), jnp.float32),
          pltpu.VMEM_SHARED((NPAD, 16), jnp.float32),
          *([pltpu.SemaphoreType.DMA] * NBUF),
      ],
  )
  def deg(ones_hbm, zeros_hbm, dst_hbm, out_hbm, dst_v, ones_v, buf_v,
          acc_sh, *ss):
    c = lax.axis_index("c")
    s = lax.axis_index("s")
    w = c * NSUB + s

    pltpu.sync_copy(zeros_hbm, buf_v)
    row0 = s * ROWS_PER_SUB
    for k in range(ROWS_PER_SUB // C):
      pltpu.sync_copy(buf_v, acc_sh.at[pl.ds(row0 + k * C, C)])
    pltpu.sync_copy(ones_hbm, ones_v)
    pltpu.sync_copy(dst_hbm.at[w], dst_v)
    plsc.subcore_barrier()

    @pl.loop(0, CHUNKS)
    def _(j):
      pltpu.sync_copy(ones_v, acc_sh.at[dst_v.at[j]], add=True)

    plsc.subcore_barrier()

    for k in range(ROWS_PER_SUB // C):
      sl = pl.ds(row0 + k * C, C)
      pltpu.sync_copy(acc_sh.at[sl], buf_v)
      pltpu.sync_copy(buf_v, out_hbm.at[c, sl])

  return deg


_sc_agg = _make_sc_agg(HID)
_sc_agg2 = _make_sc_agg_colsplit(HID)
_sc_deg = _make_sc_deg()


_BM = 1024
_GRID = NPAD // _BM


def _mm_body(x_ref, w_ref, o_ref):
  o_ref[...] = jnp.dot(x_ref[...], w_ref[...],
                       preferred_element_type=jnp.float32)


def _tc_matmul(x, w):
  m, k = x.shape
  n = w.shape[1]
  return pl.pallas_call(
      _mm_body,
      grid=(m // _BM,),
      in_specs=[pl.BlockSpec((_BM, k), lambda i: (i, 0)),
                pl.BlockSpec((k, n), lambda i: (0, 0))],
      out_specs=pl.BlockSpec((_BM, n), lambda i: (i, 0)),
      out_shape=jax.ShapeDtypeStruct((m, n), jnp.float32),
  )(x, w)


def _dinv_scale_body(degp_ref, h_ref, dinv_ref, g_ref, i_ref=None):
  del i_ref
  i = pl.program_id(0)
  deg = degp_ref[0, :, 0:1] + degp_ref[1, :, 0:1] + 1.0
  rid = lax.broadcasted_iota(jnp.int32, (_BM, 1), 0) + i * _BM
  dinv = jnp.where(rid < N, lax.rsqrt(deg), 0.0)
  dinv_ref[...] = dinv
  g_ref[...] = h_ref[...] * dinv


def _tc_dinv_scale(degp, h):
  return pl.pallas_call(
      _dinv_scale_body,
      grid=(_GRID,),
      in_specs=[pl.BlockSpec((NCORES, _BM, 16), lambda i: (0, i, 0)),
                pl.BlockSpec((_BM, HID), lambda i: (i, 0))],
      out_specs=[pl.BlockSpec((_BM, 1), lambda i: (i, 0)),
                 pl.BlockSpec((_BM, HID), lambda i: (i, 0))],
      out_shape=[jax.ShapeDtypeStruct((NPAD, 1), jnp.float32),
                 jax.ShapeDtypeStruct((NPAD, HID), jnp.float32)],
  )(degp, h)


def _mid_body(p_ref, g_ref, dinv_ref, b_ref, w_ref, g2a_ref, g2b_ref):
  acc = p_ref[0] + p_ref[1] + g_ref[...]
  z = jax.nn.relu(dinv_ref[...] * acc + b_ref[...])
  g2 = dinv_ref[...] * jnp.dot(z, w_ref[...],
                               preferred_element_type=jnp.float32)
  g2a_ref[...] = g2[:, :HID]
  g2b_ref[...] = g2[:, HID:]


def _tc_mid(p, g, dinv, b, w):
  return pl.pallas_call(
      _mid_body,
      grid=(_GRID,),
      in_specs=[pl.BlockSpec((NCORES, _BM, HID), lambda i: (0, i, 0)),
                pl.BlockSpec((_BM, HID), lambda i: (i, 0)),
                pl.BlockSpec((_BM, 1), lambda i: (i, 0)),
                pl.BlockSpec((1, HID), lambda i: (0, 0)),
                pl.BlockSpec((HID, OUT_CH), lambda i: (0, 0))],
      out_specs=[pl.BlockSpec((_BM, HID), lambda i: (i, 0)),
                 pl.BlockSpec((_BM, HID), lambda i: (i, 0))],
      out_shape=[jax.ShapeDtypeStruct((NPAD, HID), jnp.float32),
                 jax.ShapeDtypeStruct((NPAD, HID), jnp.float32)],
  )(p, g, dinv, b, w)


def _final_body(p_ref, ga_ref, gb_ref, dinv_ref, b_ref, o_ref):
  ha = p_ref[0] + ga_ref[...]
  hb = p_ref[1] + gb_ref[...]
  acc = jnp.concatenate([ha, hb], axis=1)
  o_ref[...] = jax.nn.relu(dinv_ref[...] * acc + b_ref[...])


def _tc_final(p, ga, gb, dinv, b):
  return pl.pallas_call(
      _final_body,
      grid=(_GRID,),
      in_specs=[pl.BlockSpec((NCORES, _BM, HID), lambda i: (0, i, 0)),
                pl.BlockSpec((_BM, HID), lambda i: (i, 0)),
                pl.BlockSpec((_BM, HID), lambda i: (i, 0)),
                pl.BlockSpec((_BM, 1), lambda i: (i, 0)),
                pl.BlockSpec((1, OUT_CH), lambda i: (0, 0))],
      out_specs=pl.BlockSpec((_BM, OUT_CH), lambda i: (i, 0)),
      out_shape=jax.ShapeDtypeStruct((NPAD, OUT_CH), jnp.float32),
  )(p, ga, gb, dinv, b)


def kernel(x, edge_index, W1, b1, W2, b2):
  src = edge_index[0].astype(jnp.int32)
  dst = edge_index[1].astype(jnp.int32)
  pad = jnp.full((EPAD - E,), N, jnp.int32)
  src_p = jnp.concatenate([src, pad]).reshape(NTILES, CHUNKS, C)
  dst_p = jnp.concatenate([dst, pad]).reshape(NTILES, CHUNKS, C)
  src_p2 = src_p.reshape(NSUB, CHUNKS2, C)
  dst_p2 = dst_p.reshape(NSUB, CHUNKS2, C)
  x_pad = jnp.zeros((NPAD, IN_CH), jnp.float32).at[:N].set(x)
  z16 = jnp.zeros((C, 16), jnp.float32)
  z64 = jnp.zeros((C, HID), jnp.float32)
  ones16 = jnp.ones((C, 16), jnp.float32)

  degp = _sc_deg(ones16, z16, dst_p)
  h1 = _tc_matmul(x_pad, W1)

  dinv, g1 = _tc_dinv_scale(degp, h1)
  p1 = _sc_agg(g1, src_p, dst_p, z64)
  g2a, g2b = _tc_mid(p1, g1, dinv, b1.reshape(1, HID), W2)
  p2 = _sc_agg2(g2a, g2b, src_p2, dst_p2, z64)
  out = _tc_final(p2, g2a, g2b, dinv, b2.reshape(1, OUT_CH))
  return out[:N]


# speedup vs baseline: 2.0934x; 1.0693x over previous
import functools

import jax
import jax.numpy as jnp
from jax import lax
from jax.experimental import pallas as pl
from jax.experimental.pallas import tpu as pltpu
from jax.experimental.pallas import tpu_sc as plsc

N = 10000
NPAD = 10240
E = 320000
C = 128
NCORES = 2
NSUB = 16
NTILES = NCORES * NSUB
NBUF = 4
CHUNKS = 80
EPAD = NTILES * CHUNKS * C
ROWS_PER_SUB = NPAD // NSUB
IN_CH, HID, OUT_CH = 128, 64, 128


def _make_sc_agg(d):
  mesh = plsc.VectorSubcoreMesh(core_axis_name="c", subcore_axis_name="s")

  @functools.partial(
      pl.kernel,
      out_type=jax.ShapeDtypeStruct((NCORES, NPAD, d), jnp.float32),
      mesh=mesh,
      compiler_params=pltpu.CompilerParams(use_tc_tiling_on_sc=False),
      scratch_types=[
          pltpu.VMEM((CHUNKS, C), jnp.int32),
          pltpu.VMEM((CHUNKS, C), jnp.int32),
          pltpu.VMEM((C, d), jnp.float32),
          pltpu.VMEM((C, d), jnp.float32),
          pltpu.VMEM_SHARED((NPAD, d), jnp.float32),
          pltpu.VMEM_SHARED((NPAD, d), jnp.float32),
          pltpu.SemaphoreType.DMA,
          pltpu.SemaphoreType.DMA,
      ],
  )
  def agg(table_hbm, src_hbm, dst_hbm, zeros_hbm, out_hbm,
          src_v, dst_v, buf0, buf1, table_sh, acc_sh, sg0, sg1):
    bufs, sg = (buf0, buf1), (sg0, sg1)
    buf_v = buf0
    c = lax.axis_index("c")
    s = lax.axis_index("s")
    w = c * NSUB + s
    row0 = s * ROWS_PER_SUB

    pltpu.sync_copy(zeros_hbm, buf_v)
    for k in range(ROWS_PER_SUB // C):
      pltpu.sync_copy(buf_v, acc_sh.at[pl.ds(row0 + k * C, C)])
    for k in range(ROWS_PER_SUB // C):
      sl = pl.ds(row0 + k * C, C)
      pltpu.sync_copy(table_hbm.at[sl], buf_v)
      pltpu.sync_copy(buf_v, table_sh.at[sl])

    pltpu.sync_copy(src_hbm.at[w], src_v)
    pltpu.sync_copy(dst_hbm.at[w], dst_v)
    plsc.subcore_barrier()

    for b in range(2):
      pltpu.async_copy(table_sh.at[src_v.at[b]], bufs[b], sg[b])

    @pl.loop(0, CHUNKS, step=2)
    def _(j):
      for b in range(2):
        jj = j + b
        pltpu.make_async_copy(table_sh.at[src_v.at[0]], bufs[b], sg[b]).wait()
        pltpu.sync_copy(bufs[b], acc_sh.at[dst_v.at[jj]], add=True)

        @pl.when(jj + 2 < CHUNKS)
        def _():
          pltpu.async_copy(table_sh.at[src_v.at[jj + 2]], bufs[b], sg[b])

    plsc.subcore_barrier()

    for k in range(ROWS_PER_SUB // C):
      sl = pl.ds(row0 + k * C, C)
      pltpu.sync_copy(acc_sh.at[sl], buf_v)
      pltpu.sync_copy(buf_v, out_hbm.at[c, sl])

  return agg


CHUNKS2 = CHUNKS * NCORES


def _make_sc_agg_colsplit(d):
  mesh = plsc.VectorSubcoreMesh(core_axis_name="c", subcore_axis_name="s")

  @functools.partial(
      pl.kernel,
      out_type=jax.ShapeDtypeStruct((NCORES, NPAD, d), jnp.float32),
      mesh=mesh,
      compiler_params=pltpu.CompilerParams(use_tc_tiling_on_sc=False),
      scratch_types=[
          pltpu.VMEM((CHUNKS2, C), jnp.int32),
          pltpu.VMEM((CHUNKS2, C), jnp.int32),
          pltpu.VMEM((C, d), jnp.float32),
          pltpu.VMEM((C, d), jnp.float32),
          pltpu.VMEM_SHARED((NPAD, d), jnp.float32),
          pltpu.VMEM_SHARED((NPAD, d), jnp.float32),
          pltpu.SemaphoreType.DMA,
          pltpu.SemaphoreType.DMA,
      ],
  )
  def agg2(ta_hbm, tb_hbm, src_hbm, dst_hbm, zeros_hbm, out_hbm,
           src_v, dst_v, buf0, buf1, table_sh, acc_sh, sg0, sg1):
    bufs, sg = (buf0, buf1), (sg0, sg1)
    buf_v = buf0
    c = lax.axis_index("c")
    s = lax.axis_index("s")
    row0 = s * ROWS_PER_SUB

    pltpu.sync_copy(zeros_hbm, buf_v)
    for k in range(ROWS_PER_SUB // C):
      pltpu.sync_copy(buf_v, acc_sh.at[pl.ds(row0 + k * C, C)])
    for k in range(ROWS_PER_SUB // C):
      sl = pl.ds(row0 + k * C, C)

      @pl.when(c == 0)
      def _():
        pltpu.sync_copy(ta_hbm.at[sl], buf_v)

      @pl.when(c == 1)
      def _():
        pltpu.sync_copy(tb_hbm.at[sl], buf_v)

      pltpu.sync_copy(buf_v, table_sh.at[sl])

    pltpu.sync_copy(src_hbm.at[s], src_v)
    pltpu.sync_copy(dst_hbm.at[s], dst_v)
    plsc.subcore_barrier()

    @pl.loop(0, CHUNKS2)
    def _(j):
      pltpu.sync_copy(table_sh.at[src_v.at[j]], buf_v)
      pltpu.sync_copy(buf_v, acc_sh.at[dst_v.at[j]], add=True)

    plsc.subcore_barrier()

    for k in range(ROWS_PER_SUB // C):
      sl = pl.ds(row0 + k * C, C)
      pltpu.sync_copy(acc_sh.at[sl], buf_v)
      pltpu.sync_copy(buf_v, out_hbm.at[c, sl])

  return agg2


def _make_sc_deg():
  mesh = plsc.VectorSubcoreMesh(core_axis_name="c", subcore_axis_name="s")

  @functools.partial(
      pl.kernel,
      out_type=jax.ShapeDtypeStruct((NCORES, NPAD, 16), jnp.float32),
      mesh=mesh,
      compiler_params=pltpu.CompilerParams(use_tc_tiling_on_sc=False),
      scratch_types=[
          pltpu.VMEM((CHUNKS, C), jnp.int32),
          pltpu.VMEM((C, 16), jnp.float32),
          pltpu.VMEM((C, 16), jnp.float32),
          pltpu.VMEM_SHARED((NPAD, 16), jnp.float32),
          *([pltpu.SemaphoreType.DMA] * NBUF),
      ],
  )
  def deg(ones_hbm, zeros_hbm, dst_hbm, out_hbm, dst_v, ones_v, buf_v,
          acc_sh, *ss):
    c = lax.axis_index("c")
    s = lax.axis_index("s")
    w = c * NSUB + s

    pltpu.sync_copy(zeros_hbm, buf_v)
    row0 = s * ROWS_PER_SUB
    for k in range(ROWS_PER_SUB // C):
      pltpu.sync_copy(buf_v, acc_sh.at[pl.ds(row0 + k * C, C)])
    pltpu.sync_copy(ones_hbm, ones_v)
    pltpu.sync_copy(dst_hbm.at[w], dst_v)
    plsc.subcore_barrier()

    @pl.loop(0, CHUNKS)
    def _(j):
      pltpu.sync_copy(ones_v, acc_sh.at[dst_v.at[j]], add=True)

    plsc.subcore_barrier()

    for k in range(ROWS_PER_SUB // C):
      sl = pl.ds(row0 + k * C, C)
      pltpu.sync_copy(acc_sh.at[sl], buf_v)
      pltpu.sync_copy(buf_v, out_hbm.at[c, sl])

  return deg


_sc_agg = _make_sc_agg(HID)
_sc_agg2 = _make_sc_agg_colsplit(HID)
_sc_deg = _make_sc_deg()


_BM = 1024
_GRID = NPAD // _BM


def _mm_body(x_ref, w_ref, o_ref):
  o_ref[...] = jnp.dot(x_ref[...], w_ref[...],
                       preferred_element_type=jnp.float32)


def _tc_matmul(x, w):
  m, k = x.shape
  n = w.shape[1]
  return pl.pallas_call(
      _mm_body,
      grid=(m // _BM,),
      in_specs=[pl.BlockSpec((_BM, k), lambda i: (i, 0)),
                pl.BlockSpec((k, n), lambda i: (0, 0))],
      out_specs=pl.BlockSpec((_BM, n), lambda i: (i, 0)),
      out_shape=jax.ShapeDtypeStruct((m, n), jnp.float32),
  )(x, w)


def _dinv_scale_body(degp_ref, h_ref, dinv_ref, g_ref, i_ref=None):
  del i_ref
  i = pl.program_id(0)
  deg = degp_ref[0, :, 0:1] + degp_ref[1, :, 0:1] + 1.0
  rid = lax.broadcasted_iota(jnp.int32, (_BM, 1), 0) + i * _BM
  dinv = jnp.where(rid < N, lax.rsqrt(deg), 0.0)
  dinv_ref[...] = dinv
  g_ref[...] = h_ref[...] * dinv


def _tc_dinv_scale(degp, h):
  return pl.pallas_call(
      _dinv_scale_body,
      grid=(_GRID,),
      in_specs=[pl.BlockSpec((NCORES, _BM, 16), lambda i: (0, i, 0)),
                pl.BlockSpec((_BM, HID), lambda i: (i, 0))],
      out_specs=[pl.BlockSpec((_BM, 1), lambda i: (i, 0)),
                 pl.BlockSpec((_BM, HID), lambda i: (i, 0))],
      out_shape=[jax.ShapeDtypeStruct((NPAD, 1), jnp.float32),
                 jax.ShapeDtypeStruct((NPAD, HID), jnp.float32)],
  )(degp, h)


def _mid_body(p_ref, g_ref, dinv_ref, b_ref, w_ref, g2a_ref, g2b_ref):
  acc = p_ref[0] + p_ref[1] + g_ref[...]
  z = jax.nn.relu(dinv_ref[...] * acc + b_ref[...])
  g2 = dinv_ref[...] * jnp.dot(z, w_ref[...],
                               preferred_element_type=jnp.float32)
  g2a_ref[...] = g2[:, :HID]
  g2b_ref[...] = g2[:, HID:]


def _tc_mid(p, g, dinv, b, w):
  return pl.pallas_call(
      _mid_body,
      grid=(_GRID,),
      in_specs=[pl.BlockSpec((NCORES, _BM, HID), lambda i: (0, i, 0)),
                pl.BlockSpec((_BM, HID), lambda i: (i, 0)),
                pl.BlockSpec((_BM, 1), lambda i: (i, 0)),
                pl.BlockSpec((1, HID), lambda i: (0, 0)),
                pl.BlockSpec((HID, OUT_CH), lambda i: (0, 0))],
      out_specs=[pl.BlockSpec((_BM, HID), lambda i: (i, 0)),
                 pl.BlockSpec((_BM, HID), lambda i: (i, 0))],
      out_shape=[jax.ShapeDtypeStruct((NPAD, HID), jnp.float32),
                 jax.ShapeDtypeStruct((NPAD, HID), jnp.float32)],
  )(p, g, dinv, b, w)


def _final_body(p_ref, ga_ref, gb_ref, dinv_ref, b_ref, o_ref):
  ha = p_ref[0] + ga_ref[...]
  hb = p_ref[1] + gb_ref[...]
  acc = jnp.concatenate([ha, hb], axis=1)
  o_ref[...] = jax.nn.relu(dinv_ref[...] * acc + b_ref[...])


def _tc_final(p, ga, gb, dinv, b):
  return pl.pallas_call(
      _final_body,
      grid=(_GRID,),
      in_specs=[pl.BlockSpec((NCORES, _BM, HID), lambda i: (0, i, 0)),
                pl.BlockSpec((_BM, HID), lambda i: (i, 0)),
                pl.BlockSpec((_BM, HID), lambda i: (i, 0)),
                pl.BlockSpec((_BM, 1), lambda i: (i, 0)),
                pl.BlockSpec((1, OUT_CH), lambda i: (0, 0))],
      out_specs=pl.BlockSpec((_BM, OUT_CH), lambda i: (i, 0)),
      out_shape=jax.ShapeDtypeStruct((NPAD, OUT_CH), jnp.float32),
  )(p, ga, gb, dinv, b)


def kernel(x, edge_index, W1, b1, W2, b2):
  src = edge_index[0].astype(jnp.int32)
  dst = edge_index[1].astype(jnp.int32)
  pad = jnp.full((EPAD - E,), N, jnp.int32)
  src_p = jnp.concatenate([src, pad]).reshape(NTILES, CHUNKS, C)
  dst_p = jnp.concatenate([dst, pad]).reshape(NTILES, CHUNKS, C)
  src_p2 = src_p.reshape(NSUB, CHUNKS2, C)
  dst_p2 = dst_p.reshape(NSUB, CHUNKS2, C)
  x_pad = jnp.zeros((NPAD, IN_CH), jnp.float32).at[:N].set(x)
  z16 = jnp.zeros((C, 16), jnp.float32)
  z64 = jnp.zeros((C, HID), jnp.float32)
  ones16 = jnp.ones((C, 16), jnp.float32)

  degp = _sc_deg(ones16, z16, dst_p)
  h1 = _tc_matmul(x_pad, W1)

  dinv, g1 = _tc_dinv_scale(degp, h1)
  p1 = _sc_agg(g1, src_p, dst_p, z64)
  g2a, g2b = _tc_mid(p1, g1, dinv, b1.reshape(1, HID), W2)
  p2 = _sc_agg2(g2a, g2b, src_p2, dst_p2, z64)
  out = _tc_final(p2, g2a, g2b, dinv, b2.reshape(1, OUT_CH))
  return out[:N]


# speedup vs baseline: 2.3017x; 1.0995x over previous
import functools

import jax
import jax.numpy as jnp
from jax import lax
from jax.experimental import pallas as pl
from jax.experimental.pallas import tpu as pltpu
from jax.experimental.pallas import tpu_sc as plsc

N = 10000
NPAD = 10240
E = 320000
C = 128
NCORES = 2
NSUB = 16
NTILES = NCORES * NSUB
NBUF = 4
CHUNKS = 80
EPAD = NTILES * CHUNKS * C
ROWS_PER_SUB = NPAD // NSUB
IN_CH, HID, OUT_CH = 128, 64, 128


def _make_sc_agg(d):
  mesh = plsc.VectorSubcoreMesh(core_axis_name="c", subcore_axis_name="s")

  @functools.partial(
      pl.kernel,
      out_type=jax.ShapeDtypeStruct((NCORES, NPAD, d), jnp.float32),
      mesh=mesh,
      compiler_params=pltpu.CompilerParams(use_tc_tiling_on_sc=False),
      scratch_types=[
          pltpu.VMEM((CHUNKS, C), jnp.int32),
          pltpu.VMEM((CHUNKS, C), jnp.int32),
          pltpu.VMEM((C, d), jnp.float32),
          pltpu.VMEM((C, d), jnp.float32),
          pltpu.VMEM_SHARED((NPAD, d), jnp.float32),
          pltpu.VMEM_SHARED((NPAD, d), jnp.float32),
          pltpu.SemaphoreType.DMA,
          pltpu.SemaphoreType.DMA,
      ],
  )
  def agg(table_hbm, src_hbm, dst_hbm, zeros_hbm, out_hbm,
          src_v, dst_v, buf0, buf1, table_sh, acc_sh, sg0, sg1):
    bufs, sg = (buf0, buf1), (sg0, sg1)
    buf_v = buf0
    c = lax.axis_index("c")
    s = lax.axis_index("s")
    w = c * NSUB + s
    row0 = s * ROWS_PER_SUB

    pltpu.sync_copy(zeros_hbm, buf_v)
    for k in range(ROWS_PER_SUB // C):
      pltpu.sync_copy(buf_v, acc_sh.at[pl.ds(row0 + k * C, C)])
    for k in range(ROWS_PER_SUB // C):
      sl = pl.ds(row0 + k * C, C)
      pltpu.sync_copy(table_hbm.at[sl], buf_v)
      pltpu.sync_copy(buf_v, table_sh.at[sl])

    pltpu.sync_copy(src_hbm.at[w], src_v)
    pltpu.sync_copy(dst_hbm.at[w], dst_v)
    plsc.subcore_barrier()

    for b in range(2):
      pltpu.async_copy(table_sh.at[src_v.at[b]], bufs[b], sg[b])

    @pl.loop(0, CHUNKS, step=2)
    def _(j):
      for b in range(2):
        jj = j + b
        pltpu.make_async_copy(table_sh.at[src_v.at[0]], bufs[b], sg[b]).wait()
        pltpu.sync_copy(bufs[b], acc_sh.at[dst_v.at[jj]], add=True)

        @pl.when(jj + 2 < CHUNKS)
        def _():
          pltpu.async_copy(table_sh.at[src_v.at[jj + 2]], bufs[b], sg[b])

    plsc.subcore_barrier()

    for k in range(ROWS_PER_SUB // C):
      sl = pl.ds(row0 + k * C, C)
      pltpu.sync_copy(acc_sh.at[sl], buf_v)
      pltpu.sync_copy(buf_v, out_hbm.at[c, sl])

  return agg


def _make_sc_deg():
  mesh = plsc.VectorSubcoreMesh(core_axis_name="c", subcore_axis_name="s")

  @functools.partial(
      pl.kernel,
      out_type=jax.ShapeDtypeStruct((NCORES, NPAD, 16), jnp.float32),
      mesh=mesh,
      compiler_params=pltpu.CompilerParams(use_tc_tiling_on_sc=False),
      scratch_types=[
          pltpu.VMEM((CHUNKS, C), jnp.int32),
          pltpu.VMEM((C, 16), jnp.float32),
          pltpu.VMEM((C, 16), jnp.float32),
          pltpu.VMEM_SHARED((NPAD, 16), jnp.float32),
          *([pltpu.SemaphoreType.DMA] * NBUF),
      ],
  )
  def deg(ones_hbm, zeros_hbm, dst_hbm, out_hbm, dst_v, ones_v, buf_v,
          acc_sh, *ss):
    c = lax.axis_index("c")
    s = lax.axis_index("s")
    w = c * NSUB + s

    pltpu.sync_copy(zeros_hbm, buf_v)
    row0 = s * ROWS_PER_SUB
    for k in range(ROWS_PER_SUB // C):
      pltpu.sync_copy(buf_v, acc_sh.at[pl.ds(row0 + k * C, C)])
    pltpu.sync_copy(ones_hbm, ones_v)
    pltpu.sync_copy(dst_hbm.at[w], dst_v)
    plsc.subcore_barrier()

    @pl.loop(0, CHUNKS)
    def _(j):
      pltpu.sync_copy(ones_v, acc_sh.at[dst_v.at[j]], add=True)

    plsc.subcore_barrier()

    for k in range(ROWS_PER_SUB // C):
      sl = pl.ds(row0 + k * C, C)
      pltpu.sync_copy(acc_sh.at[sl], buf_v)
      pltpu.sync_copy(buf_v, out_hbm.at[c, sl])

  return deg


_sc_agg = _make_sc_agg(HID)
_sc_deg = _make_sc_deg()


_BM = 1024
_GRID = NPAD // _BM


def _prep_body(degp_ref, x_ref, w_ref, dinv_ref, g_ref):
  i = pl.program_id(0)
  deg = degp_ref[0, :, 0:1] + degp_ref[1, :, 0:1] + 1.0
  rid = lax.broadcasted_iota(jnp.int32, (_BM, 1), 0) + i * _BM
  dinv = jnp.where(rid < N, lax.rsqrt(deg), 0.0)
  dinv_ref[...] = dinv
  h = jnp.dot(x_ref[...], w_ref[...], preferred_element_type=jnp.float32)
  g_ref[...] = h * dinv


def _tc_prep(degp, x, w):
  return pl.pallas_call(
      _prep_body,
      grid=(_GRID,),
      in_specs=[pl.BlockSpec((NCORES, _BM, 16), lambda i: (0, i, 0)),
                pl.BlockSpec((_BM, IN_CH), lambda i: (i, 0)),
                pl.BlockSpec((IN_CH, HID), lambda i: (0, 0))],
      out_specs=[pl.BlockSpec((_BM, 1), lambda i: (i, 0)),
                 pl.BlockSpec((_BM, HID), lambda i: (i, 0))],
      out_shape=[jax.ShapeDtypeStruct((NPAD, 1), jnp.float32),
                 jax.ShapeDtypeStruct((NPAD, HID), jnp.float32)],
  )(degp, x, w)


def _mid_body(p_ref, g_ref, dinv_ref, b_ref, w_ref, g2a_ref, g2b_ref):
  acc = p_ref[0] + p_ref[1] + g_ref[...]
  z = jax.nn.relu(dinv_ref[...] * acc + b_ref[...])
  g2 = dinv_ref[...] * jnp.dot(z, w_ref[...],
                               preferred_element_type=jnp.float32)
  g2a_ref[...] = g2[:, :HID]
  g2b_ref[...] = g2[:, HID:]


def _tc_mid(p, g, dinv, b, w):
  return pl.pallas_call(
      _mid_body,
      grid=(_GRID,),
      in_specs=[pl.BlockSpec((NCORES, _BM, HID), lambda i: (0, i, 0)),
                pl.BlockSpec((_BM, HID), lambda i: (i, 0)),
                pl.BlockSpec((_BM, 1), lambda i: (i, 0)),
                pl.BlockSpec((1, HID), lambda i: (0, 0)),
                pl.BlockSpec((HID, OUT_CH), lambda i: (0, 0))],
      out_specs=[pl.BlockSpec((_BM, HID), lambda i: (i, 0)),
                 pl.BlockSpec((_BM, HID), lambda i: (i, 0))],
      out_shape=[jax.ShapeDtypeStruct((NPAD, HID), jnp.float32),
                 jax.ShapeDtypeStruct((NPAD, HID), jnp.float32)],
  )(p, g, dinv, b, w)


def _final_body(pa_ref, pb_ref, ga_ref, gb_ref, dinv_ref, b_ref, o_ref):
  ha = pa_ref[0] + pa_ref[1] + ga_ref[...]
  hb = pb_ref[0] + pb_ref[1] + gb_ref[...]
  acc = jnp.concatenate([ha, hb], axis=1)
  o_ref[...] = jax.nn.relu(dinv_ref[...] * acc + b_ref[...])


def _tc_final(pa, pb, ga, gb, dinv, b):
  return pl.pallas_call(
      _final_body,
      grid=(_GRID,),
      in_specs=[pl.BlockSpec((NCORES, _BM, HID), lambda i: (0, i, 0)),
                pl.BlockSpec((NCORES, _BM, HID), lambda i: (0, i, 0)),
                pl.BlockSpec((_BM, HID), lambda i: (i, 0)),
                pl.BlockSpec((_BM, HID), lambda i: (i, 0)),
                pl.BlockSpec((_BM, 1), lambda i: (i, 0)),
                pl.BlockSpec((1, OUT_CH), lambda i: (0, 0))],
      out_specs=pl.BlockSpec((_BM, OUT_CH), lambda i: (i, 0)),
      out_shape=jax.ShapeDtypeStruct((NPAD, OUT_CH), jnp.float32),
  )(pa, pb, ga, gb, dinv, b)


def kernel(x, edge_index, W1, b1, W2, b2):
  src = edge_index[0].astype(jnp.int32)
  dst = edge_index[1].astype(jnp.int32)
  pad = jnp.full((EPAD - E,), N, jnp.int32)
  src_p = jnp.concatenate([src, pad]).reshape(NTILES, CHUNKS, C)
  dst_p = jnp.concatenate([dst, pad]).reshape(NTILES, CHUNKS, C)
  x_pad = jnp.zeros((NPAD, IN_CH), jnp.float32).at[:N].set(x)
  z16 = jnp.zeros((C, 16), jnp.float32)
  z64 = jnp.zeros((C, HID), jnp.float32)
  ones16 = jnp.ones((C, 16), jnp.float32)

  degp = _sc_deg(ones16, z16, dst_p)
  dinv, g1 = _tc_prep(degp, x_pad, W1)
  p1 = _sc_agg(g1, src_p, dst_p, z64)
  g2a, g2b = _tc_mid(p1, g1, dinv, b1.reshape(1, HID), W2)
  p2a = _sc_agg(g2a, src_p, dst_p, z64)
  p2b = _sc_agg(g2b, src_p, dst_p, z64)
  out = _tc_final(p2a, p2b, g2a, g2b, dinv, b2.reshape(1, OUT_CH))
  return out[:N]


# speedup vs baseline: 2.3036x; 1.0008x over previous
import functools

import jax
import jax.numpy as jnp
from jax import lax
from jax.experimental import pallas as pl
from jax.experimental.pallas import tpu as pltpu
from jax.experimental.pallas import tpu_sc as plsc

N = 10000
NPAD = 10240
E = 320000
C = 128
NCORES = 2
NSUB = 16
NTILES = NCORES * NSUB
NBUF = 4
CHUNKS = 80
EPAD = NTILES * CHUNKS * C
ROWS_PER_SUB = NPAD // NSUB
IN_CH, HID, OUT_CH = 128, 64, 128


def _make_sc_agg(d):
  mesh = plsc.VectorSubcoreMesh(core_axis_name="c", subcore_axis_name="s")

  @functools.partial(
      pl.kernel,
      out_type=jax.ShapeDtypeStruct((NCORES, NPAD, d), jnp.float32),
      mesh=mesh,
      compiler_params=pltpu.CompilerParams(use_tc_tiling_on_sc=False),
      scratch_types=[
          pltpu.VMEM((CHUNKS, C), jnp.int32),
          pltpu.VMEM((CHUNKS, C), jnp.int32),
          pltpu.VMEM((C, d), jnp.float32),
          pltpu.VMEM((C, d), jnp.float32),
          pltpu.VMEM_SHARED((NPAD, d), jnp.float32),
          pltpu.VMEM_SHARED((NPAD, d), jnp.float32),
          pltpu.SemaphoreType.DMA,
          pltpu.SemaphoreType.DMA,
      ],
  )
  def agg(table_hbm, src_hbm, dst_hbm, zeros_hbm, out_hbm,
          src_v, dst_v, buf0, buf1, table_sh, acc_sh, sg0, sg1):
    bufs, sg = (buf0, buf1), (sg0, sg1)
    buf_v = buf0
    c = lax.axis_index("c")
    s = lax.axis_index("s")
    w = c * NSUB + s
    row0 = s * ROWS_PER_SUB

    pltpu.sync_copy(zeros_hbm, buf_v)
    for k in range(ROWS_PER_SUB // C):
      pltpu.sync_copy(buf_v, acc_sh.at[pl.ds(row0 + k * C, C)])
    for k in range(ROWS_PER_SUB // C):
      sl = pl.ds(row0 + k * C, C)
      pltpu.sync_copy(table_hbm.at[sl], buf_v)
      pltpu.sync_copy(buf_v, table_sh.at[sl])

    pltpu.sync_copy(src_hbm.at[w], src_v)
    pltpu.sync_copy(dst_hbm.at[w], dst_v)
    plsc.subcore_barrier()

    for b in range(2):
      pltpu.async_copy(table_sh.at[src_v.at[b]], bufs[b], sg[b])

    @pl.loop(0, CHUNKS, step=2)
    def _(j):
      for b in range(2):
        jj = j + b
        pltpu.make_async_copy(table_sh.at[src_v.at[0]], bufs[b], sg[b]).wait()
        pltpu.sync_copy(bufs[b], acc_sh.at[dst_v.at[jj]], add=True)

        @pl.when(jj + 2 < CHUNKS)
        def _():
          pltpu.async_copy(table_sh.at[src_v.at[jj + 2]], bufs[b], sg[b])

    plsc.subcore_barrier()

    for k in range(ROWS_PER_SUB // C):
      sl = pl.ds(row0 + k * C, C)
      pltpu.sync_copy(acc_sh.at[sl], buf_v)
      pltpu.sync_copy(buf_v, out_hbm.at[c, sl])

  return agg


def _make_sc_agg_dual(d):
  mesh = plsc.VectorSubcoreMesh(core_axis_name="c", subcore_axis_name="s")

  @functools.partial(
      pl.kernel,
      out_type=(jax.ShapeDtypeStruct((NCORES, NPAD, d), jnp.float32),
                jax.ShapeDtypeStruct((NCORES, NPAD, d), jnp.float32)),
      mesh=mesh,
      compiler_params=pltpu.CompilerParams(use_tc_tiling_on_sc=False),
      scratch_types=[
          pltpu.VMEM((CHUNKS, C), jnp.int32),
          pltpu.VMEM((CHUNKS, C), jnp.int32),
          pltpu.VMEM((C, d), jnp.float32),
          pltpu.VMEM((C, d), jnp.float32),
          pltpu.VMEM_SHARED((NPAD, d), jnp.float32),
          pltpu.VMEM_SHARED((NPAD, d), jnp.float32),
          pltpu.SemaphoreType.DMA,
          pltpu.SemaphoreType.DMA,
      ],
  )
  def agg2(ta_hbm, tb_hbm, src_hbm, dst_hbm, zeros_hbm, outa_hbm, outb_hbm,
           src_v, dst_v, buf0, buf1, table_sh, acc_sh, sg0, sg1):
    bufs, sg = (buf0, buf1), (sg0, sg1)
    buf_v = buf0
    c = lax.axis_index("c")
    s = lax.axis_index("s")
    w = c * NSUB + s
    row0 = s * ROWS_PER_SUB

    pltpu.sync_copy(src_hbm.at[w], src_v)
    pltpu.sync_copy(dst_hbm.at[w], dst_v)

    for tbl_hbm, out_hbm in ((ta_hbm, outa_hbm), (tb_hbm, outb_hbm)):
      pltpu.sync_copy(zeros_hbm, buf_v)
      for k in range(ROWS_PER_SUB // C):
        pltpu.sync_copy(buf_v, acc_sh.at[pl.ds(row0 + k * C, C)])
      for k in range(ROWS_PER_SUB // C):
        sl = pl.ds(row0 + k * C, C)
        pltpu.sync_copy(tbl_hbm.at[sl], buf_v)
        pltpu.sync_copy(buf_v, table_sh.at[sl])
      plsc.subcore_barrier()

      for b in range(2):
        pltpu.async_copy(table_sh.at[src_v.at[b]], bufs[b], sg[b])

      @pl.loop(0, CHUNKS, step=2)
      def _(j):
        for b in range(2):
          jj = j + b
          pltpu.make_async_copy(table_sh.at[src_v.at[0]], bufs[b],
                                sg[b]).wait()
          pltpu.sync_copy(bufs[b], acc_sh.at[dst_v.at[jj]], add=True)

          @pl.when(jj + 2 < CHUNKS)
          def _():
            pltpu.async_copy(table_sh.at[src_v.at[jj + 2]], bufs[b], sg[b])

      plsc.subcore_barrier()

      for k in range(ROWS_PER_SUB // C):
        sl = pl.ds(row0 + k * C, C)
        pltpu.sync_copy(acc_sh.at[sl], buf_v)
        pltpu.sync_copy(buf_v, out_hbm.at[c, sl])

  return agg2


def _make_sc_deg():
  mesh = plsc.VectorSubcoreMesh(core_axis_name="c", subcore_axis_name="s")

  @functools.partial(
      pl.kernel,
      out_type=jax.ShapeDtypeStruct((NCORES, NPAD, 16), jnp.float32),
      mesh=mesh,
      compiler_params=pltpu.CompilerParams(use_tc_tiling_on_sc=False),
      scratch_types=[
          pltpu.VMEM((CHUNKS, C), jnp.int32),
          pltpu.VMEM((C, 16), jnp.float32),
          pltpu.VMEM((C, 16), jnp.float32),
          pltpu.VMEM_SHARED((NPAD, 16), jnp.float32),
          *([pltpu.SemaphoreType.DMA] * NBUF),
      ],
  )
  def deg(ones_hbm, zeros_hbm, dst_hbm, out_hbm, dst_v, ones_v, buf_v,
          acc_sh, *ss):
    c = lax.axis_index("c")
    s = lax.axis_index("s")
    w = c * NSUB + s

    pltpu.sync_copy(zeros_hbm, buf_v)
    row0 = s * ROWS_PER_SUB
    for k in range(ROWS_PER_SUB // C):
      pltpu.sync_copy(buf_v, acc_sh.at[pl.ds(row0 + k * C, C)])
    pltpu.sync_copy(ones_hbm, ones_v)
    pltpu.sync_copy(dst_hbm.at[w], dst_v)
    plsc.subcore_barrier()

    @pl.loop(0, CHUNKS, step=NBUF)
    def _(j):
      for b in range(NBUF):
        jj = j + b

        @pl.when(jj >= NBUF)
        def _():
          pltpu.make_async_copy(ones_v, acc_sh.at[dst_v.at[0]], ss[b]).wait()

        pltpu.async_copy(ones_v, acc_sh.at[dst_v.at[jj]], ss[b], add=True)

    for b in range(NBUF):
      pltpu.make_async_copy(ones_v, acc_sh.at[dst_v.at[0]], ss[b]).wait()
    plsc.subcore_barrier()

    for k in range(ROWS_PER_SUB // C):
      sl = pl.ds(row0 + k * C, C)
      pltpu.sync_copy(acc_sh.at[sl], buf_v)
      pltpu.sync_copy(buf_v, out_hbm.at[c, sl])

  return deg


_sc_agg = _make_sc_agg(HID)
_sc_agg2 = _make_sc_agg_dual(HID)
_sc_deg = _make_sc_deg()


_BM = 1024
_GRID = NPAD // _BM


def _prep_body(degp_ref, x_ref, w_ref, dinv_ref, g_ref):
  i = pl.program_id(0)
  deg = degp_ref[0, :, 0:1] + degp_ref[1, :, 0:1] + 1.0
  rid = lax.broadcasted_iota(jnp.int32, (_BM, 1), 0) + i * _BM
  dinv = jnp.where(rid < N, lax.rsqrt(deg), 0.0)
  dinv_ref[...] = dinv
  h = jnp.dot(x_ref[...], w_ref[...], preferred_element_type=jnp.float32)
  g_ref[...] = h * dinv


def _tc_prep(degp, x, w):
  return pl.pallas_call(
      _prep_body,
      grid=(_GRID,),
      in_specs=[pl.BlockSpec((NCORES, _BM, 16), lambda i: (0, i, 0)),
                pl.BlockSpec((_BM, IN_CH), lambda i: (i, 0)),
                pl.BlockSpec((IN_CH, HID), lambda i: (0, 0))],
      out_specs=[pl.BlockSpec((_BM, 1), lambda i: (i, 0)),
                 pl.BlockSpec((_BM, HID), lambda i: (i, 0))],
      out_shape=[jax.ShapeDtypeStruct((NPAD, 1), jnp.float32),
                 jax.ShapeDtypeStruct((NPAD, HID), jnp.float32)],
  )(degp, x, w)


def _mid_body(p_ref, g_ref, dinv_ref, b_ref, w_ref, g2a_ref, g2b_ref):
  acc = p_ref[0] + p_ref[1] + g_ref[...]
  z = jax.nn.relu(dinv_ref[...] * acc + b_ref[...])
  g2 = dinv_ref[...] * jnp.dot(z, w_ref[...],
                               preferred_element_type=jnp.float32)
  g2a_ref[...] = g2[:, :HID]
  g2b_ref[...] = g2[:, HID:]


def _tc_mid(p, g, dinv, b, w):
  return pl.pallas_call(
      _mid_body,
      grid=(_GRID,),
      in_specs=[pl.BlockSpec((NCORES, _BM, HID), lambda i: (0, i, 0)),
                pl.BlockSpec((_BM, HID), lambda i: (i, 0)),
                pl.BlockSpec((_BM, 1), lambda i: (i, 0)),
                pl.BlockSpec((1, HID), lambda i: (0, 0)),
                pl.BlockSpec((HID, OUT_CH), lambda i: (0, 0))],
      out_specs=[pl.BlockSpec((_BM, HID), lambda i: (i, 0)),
                 pl.BlockSpec((_BM, HID), lambda i: (i, 0))],
      out_shape=[jax.ShapeDtypeStruct((NPAD, HID), jnp.float32),
                 jax.ShapeDtypeStruct((NPAD, HID), jnp.float32)],
  )(p, g, dinv, b, w)


def _final_body(pa_ref, pb_ref, ga_ref, gb_ref, dinv_ref, b_ref, o_ref):
  ha = pa_ref[0] + pa_ref[1] + ga_ref[...]
  hb = pb_ref[0] + pb_ref[1] + gb_ref[...]
  acc = jnp.concatenate([ha, hb], axis=1)
  o_ref[...] = jax.nn.relu(dinv_ref[...] * acc + b_ref[...])


def _tc_final(pa, pb, ga, gb, dinv, b):
  return pl.pallas_call(
      _final_body,
      grid=(_GRID,),
      in_specs=[pl.BlockSpec((NCORES, _BM, HID), lambda i: (0, i, 0)),
                pl.BlockSpec((NCORES, _BM, HID), lambda i: (0, i, 0)),
                pl.BlockSpec((_BM, HID), lambda i: (i, 0)),
                pl.BlockSpec((_BM, HID), lambda i: (i, 0)),
                pl.BlockSpec((_BM, 1), lambda i: (i, 0)),
                pl.BlockSpec((1, OUT_CH), lambda i: (0, 0))],
      out_specs=pl.BlockSpec((_BM, OUT_CH), lambda i: (i, 0)),
      out_shape=jax.ShapeDtypeStruct((NPAD, OUT_CH), jnp.float32),
  )(pa, pb, ga, gb, dinv, b)


def kernel(x, edge_index, W1, b1, W2, b2):
  src = edge_index[0].astype(jnp.int32)
  dst = edge_index[1].astype(jnp.int32)
  pad = jnp.full((EPAD - E,), N, jnp.int32)
  src_p = jnp.concatenate([src, pad]).reshape(NTILES, CHUNKS, C)
  dst_p = jnp.concatenate([dst, pad]).reshape(NTILES, CHUNKS, C)
  x_pad = jnp.zeros((NPAD, IN_CH), jnp.float32).at[:N].set(x)
  z16 = jnp.zeros((C, 16), jnp.float32)
  z64 = jnp.zeros((C, HID), jnp.float32)
  ones16 = jnp.ones((C, 16), jnp.float32)

  degp = _sc_deg(ones16, z16, dst_p)
  dinv, g1 = _tc_prep(degp, x_pad, W1)
  p1 = _sc_agg(g1, src_p, dst_p, z64)
  g2a, g2b = _tc_mid(p1, g1, dinv, b1.reshape(1, HID), W2)
  p2a, p2b = _sc_agg2(g2a, g2b, src_p, dst_p, z64)
  out = _tc_final(p2a, p2b, g2a, g2b, dinv, b2.reshape(1, OUT_CH))
  return out[:N]
